# baseline (speedup 1.0000x reference)
import functools

import jax
import jax.numpy as jnp
import numpy as np
from jax import lax
from jax.experimental import pallas as pl
from jax.experimental.pallas import tpu as pltpu
from jax.experimental.pallas import tpu_sc as plsc

SEQ = 200
DIM = 64
NUM_WORKERS = 32
CHUNK = 128


def _pos_encoding(length, dim):
    pos = np.arange(length)[:, np.newaxis]
    i = np.arange(dim)[np.newaxis, :]
    angle_rates = 1.0 / np.power(10000, 2 * (i // 2) / np.float32(dim))
    angle_rads = pos * angle_rates
    angle_rads[:, 0::2] = np.sin(angle_rads[:, 0::2])
    angle_rads[:, 1::2] = np.cos(angle_rads[:, 1::2])
    return jnp.asarray(angle_rads, dtype=jnp.float32)


def _sc_body(idx_hbm, pos_hbm, table_hbm, out_hbm, idx_v, pos_v, rows_v, sem):
    n_chunks = idx_hbm.shape[1]
    wid = lax.axis_index("s") * 2 + lax.axis_index("c")
    base_row = wid * (n_chunks * CHUNK)

    pltpu.sync_copy(idx_hbm.at[wid], idx_v)
    pltpu.sync_copy(pos_hbm, pos_v)

    def chunk_body(j, carry):
        pltpu.async_copy(table_hbm.at[idx_v.at[j]], rows_v, sem).wait()

        poff = lax.rem(j * CHUNK, SEQ)

        def row_body(r, c):
            l = lax.rem(poff + r, SEQ)
            for q in range(DIM // 16):
                sl = pl.ds(16 * q, 16)
                v = rows_v[r, sl]
                p = pos_v[l, sl]
                rows_v[r, sl] = v * 8.0 + p
            return c

        lax.fori_loop(0, CHUNK, row_body, 0)

        pltpu.sync_copy(rows_v, out_hbm.at[pl.ds(base_row + j * CHUNK, CHUNK)])
        return carry

    lax.fori_loop(0, n_chunks, chunk_body, 0)


def kernel(inputs, table):
    batch, seq = inputs.shape
    vocab, dim = table.shape
    total = batch * seq
    rows_per_w = total // NUM_WORKERS
    n_chunks = rows_per_w // CHUNK

    idx = inputs.reshape(NUM_WORKERS, n_chunks, CHUNK)
    pos = _pos_encoding(SEQ, dim)

    mesh = plsc.VectorSubcoreMesh(core_axis_name="c", subcore_axis_name="s")
    f = functools.partial(
        pl.kernel,
        mesh=mesh,
        out_type=jax.ShapeDtypeStruct((total, dim), jnp.float32),
        compiler_params=pltpu.CompilerParams(use_tc_tiling_on_sc=False),
        scratch_types=[
            pltpu.VMEM((n_chunks, CHUNK), jnp.int32),
            pltpu.VMEM((SEQ, dim), jnp.float32),
            pltpu.VMEM((CHUNK, dim), jnp.float32),
            pltpu.SemaphoreType.DMA,
        ],
    )(_sc_body)
    out = f(idx, pos, table)
    return out.reshape(batch, seq, dim)


# speedup vs baseline: 1.1878x; 1.1878x over previous
import functools

import jax
import jax.numpy as jnp
import numpy as np
from jax import lax
from jax.experimental import pallas as pl
from jax.experimental.pallas import tpu as pltpu
from jax.experimental.pallas import tpu_sc as plsc

SEQ = 200
DIM = 64
NUM_WORKERS = 32
CHUNK = 128
NBUF = 5
PREFETCH = 2


def _pos_encoding(length, dim):
    pos = np.arange(length)[:, np.newaxis]
    i = np.arange(dim)[np.newaxis, :]
    angle_rates = 1.0 / np.power(10000, 2 * (i // 2) / np.float32(dim))
    angle_rads = pos * angle_rates
    angle_rads[:, 0::2] = np.sin(angle_rads[:, 0::2])
    angle_rads[:, 1::2] = np.cos(angle_rads[:, 1::2])
    return jnp.asarray(angle_rads, dtype=jnp.float32)


def _sc_body(idx_hbm, oidx_hbm, pos_hbm, table_hbm, out_hbm,
             idx_v, oidx_v, pos_v, rows_v, *sems):
    gsems, ssems = sems[:NBUF], sems[NBUF:]
    n_chunks = idx_hbm.shape[1]
    wid = lax.axis_index("s") * 2 + lax.axis_index("c")

    pltpu.sync_copy(idx_hbm.at[wid], idx_v)
    pltpu.sync_copy(oidx_hbm.at[wid], oidx_v)
    pltpu.sync_copy(pos_hbm, pos_v)

    def start_gather(j, b):
        pltpu.async_copy(table_hbm.at[idx_v.at[j]], rows_v.at[b], gsems[b])

    for j in range(PREFETCH):
        start_gather(j, j)

    def body(g, carry):
        for b in range(NBUF):
            j = g * NBUF + b
            bn = (b + PREFETCH) % NBUF

            @pl.when(j + PREFETCH < n_chunks)
            def _():
                @pl.when(j + PREFETCH >= NBUF)
                def _():
                    pltpu.make_async_copy(
                        table_hbm.at[pl.ds(0, CHUNK)], rows_v.at[bn],
                        ssems[bn]).wait()
                start_gather(j + PREFETCH, bn)

            pltpu.make_async_copy(
                table_hbm.at[pl.ds(0, CHUNK)], rows_v.at[b], gsems[b]).wait()

            l = (wid * n_chunks + j) // (1024 // CHUNK)
            p0 = pos_v[l, pl.ds(0, 16)]
            p1 = pos_v[l, pl.ds(16, 16)]
            p2 = pos_v[l, pl.ds(32, 16)]
            p3 = pos_v[l, pl.ds(48, 16)]

            def row_body(r, c, _b=b, _p=(p0, p1, p2, p3)):
                for q in range(DIM // 16):
                    sl = pl.ds(16 * q, 16)
                    rows_v[_b, r, sl] = rows_v[_b, r, sl] * 8.0 + _p[q]
                return c

            lax.fori_loop(0, CHUNK, row_body, 0, unroll=4)

            pltpu.async_copy(rows_v.at[b], out_hbm.at[oidx_v.at[j]], ssems[b])
        return carry

    lax.fori_loop(0, n_chunks // NBUF, body, 0)

    for b in range(NBUF):
        pltpu.make_async_copy(
            table_hbm.at[pl.ds(0, CHUNK)], rows_v.at[b], ssems[b]).wait()


def kernel(inputs, table):
    batch, seq = inputs.shape
    vocab, dim = table.shape
    total = batch * seq
    kblocks = batch // CHUNK
    n_chunks_total = seq * kblocks
    per_w = n_chunks_total // NUM_WORKERS

    idx = inputs.T.reshape(NUM_WORKERS, per_w, CHUNK)
    c = jnp.arange(n_chunks_total, dtype=jnp.int32)[:, None]
    i = jnp.arange(CHUNK, dtype=jnp.int32)[None, :]
    oidx = (((c % kblocks) * CHUNK + i) * seq + c // kblocks).reshape(
        NUM_WORKERS, per_w, CHUNK)
    pos = _pos_encoding(SEQ, dim)

    mesh = plsc.VectorSubcoreMesh(core_axis_name="c", subcore_axis_name="s")
    f = functools.partial(
        pl.kernel,
        mesh=mesh,
        out_type=jax.ShapeDtypeStruct((total, dim), jnp.float32),
        compiler_params=pltpu.CompilerParams(use_tc_tiling_on_sc=False),
        scratch_types=[
            pltpu.VMEM((per_w, CHUNK), jnp.int32),
            pltpu.VMEM((per_w, CHUNK), jnp.int32),
            pltpu.VMEM((SEQ, dim), jnp.float32),
            pltpu.VMEM((NBUF, CHUNK, dim), jnp.float32),
        ] + [pltpu.SemaphoreType.DMA] * (2 * NBUF),
    )(_sc_body)
    out = f(idx, oidx, pos, table)
    return out.reshape(batch, seq, dim)
